# baseline (speedup 1.0000x reference)
import jax
import jax.numpy as jnp
from jax import lax
from jax.experimental import pallas as pl
from jax.experimental.pallas import tpu as pltpu

_N = 100000
_K = 256
_QMIN = 0.1
_SB = 1.0
_BN = 1000
_BIG = 3.0e9


def _seg_body(beta_ref, tid_ref, pe_ref, te_ref,
              bmax_ref, aidx_ref, cnt_ref, sums_ref):
    i = pl.program_id(0)

    @pl.when(i == 0)
    def _init():
        bmax_ref[...] = jnp.full_like(bmax_ref, -1.0)
        aidx_ref[...] = jnp.full_like(aidx_ref, _BIG)
        cnt_ref[...] = jnp.zeros_like(cnt_ref)
        sums_ref[...] = jnp.zeros_like(sums_ref)

    beta = jnp.clip(beta_ref[...], 0.0, 1.0 - 1e-4)
    tid = tid_ref[...]
    kio = lax.broadcasted_iota(jnp.int32, (1, _K), 1)
    onehot = tid == kio
    bm = jnp.where(onehot, beta, -1.0)
    blkmax = jnp.max(bm, axis=0, keepdims=True)
    cnt = jnp.sum(onehot.astype(jnp.float32), axis=0, keepdims=True)
    rowidx = (i * _BN + lax.broadcasted_iota(jnp.int32, (_BN, 1), 0)
              ).astype(jnp.float32)
    cand = jnp.where(onehot & (beta == blkmax), rowidx, _BIG)
    blkamin = jnp.min(cand, axis=0, keepdims=True)

    run_max = bmax_ref[...]
    run_idx = aidx_ref[...]
    better = blkmax > run_max
    equal = blkmax == run_max
    aidx_ref[...] = jnp.where(
        better, blkamin,
        jnp.where(equal, jnp.minimum(run_idx, blkamin), run_idx))
    bmax_ref[...] = jnp.maximum(run_max, blkmax)
    cnt_ref[...] += cnt

    is_noise = (tid == 0).astype(jnp.float32)
    pe = pe_ref[...]
    te = te_ref[...]
    eph = (pe - te) ** 2 / (te + 1.0)
    s0 = jnp.sum(is_noise * beta)
    s1 = jnp.sum(is_noise)
    s2 = jnp.sum((1.0 - is_noise) * eph)
    lane = lax.broadcasted_iota(jnp.int32, (1, 128), 1)
    sums_ref[...] += jnp.where(
        lane == 0, s0, jnp.where(lane == 1, s1,
                                 jnp.where(lane == 2, s2, 0.0)))


def _pot_body(cc_ref, beta_ref, tid_ref, ax_ref, ay_ref,
              watt_ref, wrep_ref, att_ref, rep_ref):
    i = pl.program_id(0)

    @pl.when(i == 0)
    def _init():
        att_ref[...] = jnp.zeros_like(att_ref)
        rep_ref[...] = jnp.zeros_like(rep_ref)

    xs = cc_ref[:, 0:1]
    ys = cc_ref[:, 1:2]
    beta = jnp.clip(beta_ref[...], 0.0, 1.0 - 1e-4)
    ath = 0.5 * jnp.log((1.0 + beta) / (1.0 - beta))
    q = ath * ath + _QMIN

    dx = xs - ax_ref[...]
    dy = ys - ay_ref[...]
    dsq = dx * dx + dy * dy
    dist = jnp.sqrt(dsq + 1e-6)
    hinge = jnp.maximum(0.0, 1.0 - dist)
    kio = lax.broadcasted_iota(jnp.int32, (1, _K), 1)
    onehot = tid_ref[...] == kio
    qdsq = q * dsq
    qh = q * hinge
    att_ref[...] += watt_ref[...] * jnp.sum(
        jnp.where(onehot, qdsq, 0.0), axis=0, keepdims=True)
    rep_ref[...] += wrep_ref[...] * jnp.sum(
        jnp.where(onehot, 0.0, qh), axis=0, keepdims=True)


def _seg_call(pred_beta, t_idx, pred_energy, t_energy):
    grid = (_N // _BN,)
    row_spec = pl.BlockSpec((_BN, 1), lambda i: (i, 0))
    k_spec = pl.BlockSpec((1, _K), lambda i: (0, 0))
    return pl.pallas_call(
        _seg_body,
        grid=grid,
        in_specs=[row_spec, row_spec, row_spec, row_spec],
        out_specs=[k_spec, k_spec, k_spec,
                   pl.BlockSpec((1, 128), lambda i: (0, 0))],
        out_shape=[
            jax.ShapeDtypeStruct((1, _K), jnp.float32),
            jax.ShapeDtypeStruct((1, _K), jnp.float32),
            jax.ShapeDtypeStruct((1, _K), jnp.float32),
            jax.ShapeDtypeStruct((1, 128), jnp.float32),
        ],
    )(pred_beta, t_idx, pred_energy, t_energy)


def _pot_call(pred_ccoords, pred_beta, t_idx, ax, ay, w_att, w_rep):
    grid = (_N // _BN,)
    k_spec = pl.BlockSpec((1, _K), lambda i: (0, 0))
    return pl.pallas_call(
        _pot_body,
        grid=grid,
        in_specs=[
            pl.BlockSpec((_BN, 2), lambda i: (i, 0)),
            pl.BlockSpec((_BN, 1), lambda i: (i, 0)),
            pl.BlockSpec((_BN, 1), lambda i: (i, 0)),
            k_spec, k_spec, k_spec, k_spec,
        ],
        out_specs=[k_spec, k_spec],
        out_shape=[
            jax.ShapeDtypeStruct((1, _K), jnp.float32),
            jax.ShapeDtypeStruct((1, _K), jnp.float32),
        ],
    )(pred_ccoords, pred_beta, t_idx, ax, ay, w_att, w_rep)


def kernel(pred_beta, pred_ccoords, pred_energy, t_idx, t_energy):
    bmax2, aidxf2, counts2, sums2 = _seg_call(
        pred_beta, t_idx, pred_energy, t_energy)
    bmax = bmax2[0]
    counts = counts2[0]
    aidxf = aidxf2[0]
    sums = sums2[0]

    valid = ((jnp.arange(_K) > 0) & (counts > 0)).astype(jnp.float32)
    aidx = jnp.minimum(aidxf, float(_N - 1)).astype(jnp.int32)
    ax = jnp.take(pred_ccoords[:, 0], aidx)
    ay = jnp.take(pred_ccoords[:, 1], aidx)

    bmaxc = jnp.clip(bmax, 0.0, 1.0 - 1e-4)
    athk = jnp.arctanh(bmaxc)
    q_alpha = athk * athk + _QMIN
    w_att = q_alpha * valid / (counts + 1e-3)
    w_rep = q_alpha * valid

    att_k, rep_k = _pot_call(
        pred_ccoords, pred_beta, t_idx,
        ax[None, :], ay[None, :], w_att[None, :], w_rep[None, :])

    n_obj = jnp.sum(valid) + 1e-3
    v_att = jnp.sum(att_k) / n_obj
    v_rep = jnp.sum(rep_k) / (float(_N) * n_obj)
    beta_pen = jnp.sum((1.0 - bmax) * valid) / n_obj
    s0, s1, s2 = sums[0], sums[1], sums[2]
    noise_pen = _SB * s0 / (s1 + 1e-3)
    eloss = s2 / ((float(_N) - s1) + 1e-3)
    return v_att + v_rep + beta_pen + noise_pen + eloss


# speedup vs baseline: 2.2211x; 2.2211x over previous
import functools

import jax
import jax.numpy as jnp
from jax import lax
from jax.experimental import pallas as pl
from jax.experimental.pallas import tpu as pltpu
from jax.experimental.pallas import tpu_sc as plsc

_N = 100000
_K = 256
_QMIN = 0.1
_SB = 1.0

_NP = 100352
_TILES = 16
_HPT = _NP // _TILES
_G = _HPT // 16
_BN = 2048
_NBIG = 1 << 30


def _sc_body(tid_hbm, beta_hbm, xs_hbm, ys_hbm,
             bmax_o, cnt_o, ax_o, ay_o, axn_o, ayn_o, bmn_o, cnn_o,
             tidv, betav, tbl_b, tbl_c, tbl_a,
             g_b, g_c, g_a, aclip, g_ax, g_ay,
             stage, stagei, axnv, aynv, bmnv, cnnv,
             sh_b, sh_c, sh_a, sem):
    wid = lax.axis_index("s")
    base = wid * _HPT
    lane = lax.iota(jnp.int32, 16)
    lane_off = lane * _K

    pltpu.sync_copy(tid_hbm.at[pl.ds(base, _HPT)], tidv)
    pltpu.sync_copy(beta_hbm.at[pl.ds(base, _HPT)], betav)

    def init_body(j, c):
        s = j * 16
        tbl_b[pl.ds(s, 16)] = jnp.full((16,), -1.0, jnp.float32)
        tbl_c[pl.ds(s, 16)] = jnp.zeros((16,), jnp.float32)
        tbl_a[pl.ds(s, 16)] = jnp.full((16,), _NBIG, jnp.int32)
        return c
    lax.fori_loop(0, _K, init_body, 0)

    def pass_a(i, c):
        off = i * 16
        tv = tidv[pl.ds(off, 16)]
        bv = betav[pl.ds(off, 16)]
        idx = lane_off + tv
        cb = plsc.load_gather(tbl_b, [idx])
        plsc.store_scatter(tbl_b, [idx], jnp.maximum(cb, bv))
        cc = plsc.load_gather(tbl_c, [idx])
        plsc.store_scatter(tbl_c, [idx], cc + 1.0)
        return c
    lax.fori_loop(0, _G, pass_a, 0)

    def lmerge_bc(j, c):
        s = j * 16
        accb = tbl_b[pl.ds(s, 16)]
        accc = tbl_c[pl.ds(s, 16)]
        for t in range(1, 16):
            accb = jnp.maximum(accb, tbl_b[pl.ds(t * _K + s, 16)])
            accc = accc + tbl_c[pl.ds(t * _K + s, 16)]
        g_b[pl.ds(s, 16)] = accb
        g_c[pl.ds(s, 16)] = accc
        return c
    lax.fori_loop(0, 16, lmerge_bc, 0)

    pltpu.sync_copy(g_b, sh_b.at[pl.ds(wid * _K, _K)])
    pltpu.sync_copy(g_c, sh_c.at[pl.ds(wid * _K, _K)])
    plsc.subcore_barrier()

    pltpu.sync_copy(sh_b, stage)

    def gmerge_b(j, c):
        s = j * 16
        acc = stage[pl.ds(s, 16)]
        for t in range(1, _TILES):
            acc = jnp.maximum(acc, stage[pl.ds(t * _K + s, 16)])
        g_b[pl.ds(s, 16)] = acc
        return c
    lax.fori_loop(0, 16, gmerge_b, 0)

    pltpu.sync_copy(sh_c, stage)

    def gmerge_c(j, c):
        s = j * 16
        acc = stage[pl.ds(s, 16)]
        for t in range(1, _TILES):
            acc = acc + stage[pl.ds(t * _K + s, 16)]
        g_c[pl.ds(s, 16)] = acc
        return c
    lax.fori_loop(0, 16, gmerge_c, 0)

    def pass_b1(i, c):
        off = i * 16
        tv = tidv[pl.ds(off, 16)]
        bv = betav[pl.ds(off, 16)]
        gb = plsc.load_gather(g_b, [tv])
        gidx = base + off + lane
        idx = lane_off + tv
        cur = plsc.load_gather(tbl_a, [idx])
        new = jnp.where(bv == gb, jnp.minimum(cur, gidx), cur)
        plsc.store_scatter(tbl_a, [idx], new)
        return c
    lax.fori_loop(0, _G, pass_b1, 0)

    def lmerge_a(j, c):
        s = j * 16
        acc = tbl_a[pl.ds(s, 16)]
        for t in range(1, 16):
            acc = jnp.minimum(acc, tbl_a[pl.ds(t * _K + s, 16)])
        g_a[pl.ds(s, 16)] = acc
        return c
    lax.fori_loop(0, 16, lmerge_a, 0)

    pltpu.sync_copy(g_a, sh_a.at[pl.ds(wid * _K, _K)])
    plsc.subcore_barrier()
    pltpu.sync_copy(sh_a, stagei)

    def gmerge_a(j, c):
        s = j * 16
        acc = stagei[pl.ds(s, 16)]
        for t in range(1, _TILES):
            acc = jnp.minimum(acc, stagei[pl.ds(t * _K + s, 16)])
        g_a[pl.ds(s, 16)] = acc
        aclip[pl.ds(s, 16)] = jnp.minimum(acc, _N - 1)
        return c
    lax.fori_loop(0, 16, gmerge_a, 0)

    pltpu.async_copy(xs_hbm.at[aclip], g_ax, sem).wait()
    pltpu.async_copy(ys_hbm.at[aclip], g_ay, sem).wait()

    def pass_b2(i, c):
        off = i * 16
        tv = tidv[pl.ds(off, 16)]
        axnv[pl.ds(off, 16)] = plsc.load_gather(g_ax, [tv])
        aynv[pl.ds(off, 16)] = plsc.load_gather(g_ay, [tv])
        bmnv[pl.ds(off, 16)] = plsc.load_gather(g_b, [tv])
        cnnv[pl.ds(off, 16)] = plsc.load_gather(g_c, [tv])
        return c
    lax.fori_loop(0, _G, pass_b2, 0)

    pltpu.sync_copy(axnv, axn_o.at[pl.ds(base, _HPT)])
    pltpu.sync_copy(aynv, ayn_o.at[pl.ds(base, _HPT)])
    pltpu.sync_copy(bmnv, bmn_o.at[pl.ds(base, _HPT)])
    pltpu.sync_copy(cnnv, cnn_o.at[pl.ds(base, _HPT)])

    @pl.when(wid == 0)
    def _emit_tables():
        pltpu.sync_copy(g_b, bmax_o)
        pltpu.sync_copy(g_c, cnt_o)
        pltpu.sync_copy(g_ax, ax_o)
        pltpu.sync_copy(g_ay, ay_o)


def _sc_call(tid_pad, beta_pad, xs, ys):
    f32 = jnp.float32
    kern = pl.kernel(
        _sc_body,
        out_type=[
            jax.ShapeDtypeStruct((_K,), f32),
            jax.ShapeDtypeStruct((_K,), f32),
            jax.ShapeDtypeStruct((_K,), f32),
            jax.ShapeDtypeStruct((_K,), f32),
            jax.ShapeDtypeStruct((_NP,), f32),
            jax.ShapeDtypeStruct((_NP,), f32),
            jax.ShapeDtypeStruct((_NP,), f32),
            jax.ShapeDtypeStruct((_NP,), f32),
        ],
        mesh=plsc.VectorSubcoreMesh(
            core_axis_name="c", subcore_axis_name="s",
            num_cores=1, num_subcores=_TILES),
        compiler_params=pltpu.CompilerParams(needs_layout_passes=False),
        scratch_types=[
            pltpu.VMEM((_HPT,), jnp.int32),
            pltpu.VMEM((_HPT,), f32),
            pltpu.VMEM((16 * _K,), f32),
            pltpu.VMEM((16 * _K,), f32),
            pltpu.VMEM((16 * _K,), jnp.int32),
            pltpu.VMEM((_K,), f32),
            pltpu.VMEM((_K,), f32),
            pltpu.VMEM((_K,), jnp.int32),
            pltpu.VMEM((_K,), jnp.int32),
            pltpu.VMEM((_K,), f32),
            pltpu.VMEM((_K,), f32),
            pltpu.VMEM((_TILES * _K,), f32),
            pltpu.VMEM((_TILES * _K,), jnp.int32),
            pltpu.VMEM((_HPT,), f32),
            pltpu.VMEM((_HPT,), f32),
            pltpu.VMEM((_HPT,), f32),
            pltpu.VMEM((_HPT,), f32),
            pltpu.VMEM_SHARED((_TILES * _K,), f32),
            pltpu.VMEM_SHARED((_TILES * _K,), f32),
            pltpu.VMEM_SHARED((_TILES * _K,), jnp.int32),
            pltpu.SemaphoreType.DMA,
        ],
    )
    return kern(tid_pad, beta_pad, xs, ys)


def _tc_body(cc_ref, beta_ref, tid_ref, pe_ref, te_ref, xs_ref, ys_ref,
             axn_ref, ayn_ref, bmn_ref, cnn_ref, ax_ref, ay_ref,
             rep_ref, scal_ref):
    i = pl.program_id(0)

    @pl.when(i == 0)
    def _init():
        rep_ref[...] = jnp.zeros_like(rep_ref)
        scal_ref[...] = jnp.zeros_like(scal_ref)

    gidx = i * _BN + lax.broadcasted_iota(jnp.int32, (1, _BN), 1)
    real = gidx < _N
    beta = jnp.clip(beta_ref[...], 0.0, 1.0 - 1e-4)
    ath = 0.5 * jnp.log((1.0 + beta) / (1.0 - beta))
    q = jnp.where(real, ath * ath + _QMIN, 0.0)

    xs_col = cc_ref[:, 0:1]
    ys_col = cc_ref[:, 1:2]
    dx = xs_col - ax_ref[...]
    dy = ys_col - ay_ref[...]
    dist = jnp.sqrt(dx * dx + dy * dy + 1e-6)
    hinge = jnp.maximum(0.0, 1.0 - dist)
    rep_ref[...] += jax.lax.dot_general(
        q, hinge, (((1,), (0,)), ((), ())),
        preferred_element_type=jnp.float32)

    tid = tid_ref[...]
    bmn = jnp.clip(bmn_ref[...], 0.0, 1.0 - 1e-4)
    athn = 0.5 * jnp.log((1.0 + bmn) / (1.0 - bmn))
    qan = athn * athn + _QMIN
    cnn = cnn_ref[...]
    validn = ((tid > 0) & (cnn > 0.0)).astype(jnp.float32)
    wrepn = qan * validn
    wattn = wrepn / (cnn + 1e-3)
    dxo = xs_ref[...] - axn_ref[...]
    dyo = ys_ref[...] - ayn_ref[...]
    dsq_own = dxo * dxo + dyo * dyo
    hinge_own = jnp.maximum(0.0, 1.0 - jnp.sqrt(dsq_own + 1e-6))
    att_s = jnp.sum(q * dsq_own * wattn)
    rep_own_s = jnp.sum(q * hinge_own * wrepn)

    is_noise = ((tid == 0) & real).astype(jnp.float32)
    not_noise = ((tid != 0) & real).astype(jnp.float32)
    pe = pe_ref[...]
    te = te_ref[...]
    eph = (pe - te) ** 2 / (te + 1.0)
    s0 = jnp.sum(is_noise * beta)
    s1 = jnp.sum(is_noise)
    s2 = jnp.sum(not_noise * eph)

    lanev = lax.broadcasted_iota(jnp.int32, (1, 128), 1)
    upd = jnp.where(lanev == 0, att_s,
          jnp.where(lanev == 1, rep_own_s,
          jnp.where(lanev == 2, s0,
          jnp.where(lanev == 3, s1,
          jnp.where(lanev == 4, s2, 0.0)))))
    scal_ref[...] += upd


def _tc_call(cc_pad, beta_r, tid_r, pe_r, te_r, xs_r, ys_r,
             axn_r, ayn_r, bmn_r, cnn_r, ax2, ay2):
    grid = (_NP // _BN,)
    row = pl.BlockSpec((1, _BN), lambda i: (0, i))
    kc = pl.BlockSpec((1, _K), lambda i: (0, 0))
    return pl.pallas_call(
        _tc_body,
        grid=grid,
        in_specs=[pl.BlockSpec((_BN, 2), lambda i: (i, 0)),
                  row, row, row, row, row, row, row, row, row, row,
                  kc, kc],
        out_specs=[kc, pl.BlockSpec((1, 128), lambda i: (0, 0))],
        out_shape=[
            jax.ShapeDtypeStruct((1, _K), jnp.float32),
            jax.ShapeDtypeStruct((1, 128), jnp.float32),
        ],
    )(cc_pad, beta_r, tid_r, pe_r, te_r, xs_r, ys_r,
      axn_r, ayn_r, bmn_r, cnn_r, ax2, ay2)


def kernel(pred_beta, pred_ccoords, pred_energy, t_idx, t_energy):
    f32 = jnp.float32
    pad = _NP - _N
    tid_pad = jnp.pad(t_idx[:, 0], (0, pad))
    beta_pad = jnp.pad(pred_beta[:, 0], (0, pad))
    xs = pred_ccoords[:, 0]
    ys = pred_ccoords[:, 1]

    bmax, counts, ax, ay, axn, ayn, bmn, cnn = _sc_call(
        tid_pad, beta_pad, xs, ys)

    cc_pad = jnp.pad(pred_ccoords, ((0, pad), (0, 0)))
    beta_r = beta_pad[None, :]
    tid_r = tid_pad[None, :]
    pe_r = jnp.pad(pred_energy[:, 0], (0, pad))[None, :]
    te_r = jnp.pad(t_energy[:, 0], (0, pad))[None, :]
    xs_r = jnp.pad(xs, (0, pad))[None, :]
    ys_r = jnp.pad(ys, (0, pad))[None, :]

    rep_k, scal = _tc_call(
        cc_pad, beta_r, tid_r, pe_r, te_r, xs_r, ys_r,
        axn[None, :], ayn[None, :], bmn[None, :], cnn[None, :],
        ax[None, :], ay[None, :])

    valid = ((jnp.arange(_K) > 0) & (counts > 0)).astype(f32)
    bmaxc = jnp.clip(bmax, 0.0, 1.0 - 1e-4)
    athk = jnp.arctanh(bmaxc)
    q_alpha = athk * athk + _QMIN
    w_rep = q_alpha * valid

    n_obj = jnp.sum(valid) + 1e-3
    att_s = scal[0, 0]
    rep_own_s = scal[0, 1]
    s0 = scal[0, 2]
    s1 = scal[0, 3]
    s2 = scal[0, 4]

    v_att = att_s / n_obj
    rep_total = jnp.sum(rep_k[0] * w_rep) - rep_own_s
    v_rep = rep_total / (float(_N) * n_obj)
    beta_pen = jnp.sum((1.0 - bmax) * valid) / n_obj
    noise_pen = _SB * s0 / (s1 + 1e-3)
    eloss = s2 / ((float(_N) - s1) + 1e-3)
    return v_att + v_rep + beta_pen + noise_pen + eloss


# speedup vs baseline: 2.3725x; 1.0682x over previous
import jax
import jax.numpy as jnp
from jax import lax
from jax.experimental import pallas as pl
from jax.experimental.pallas import tpu as pltpu
from jax.experimental.pallas import tpu_sc as plsc

_N = 100000
_K = 256
_QMIN = 0.1
_SB = 1.0

_NP = 100352
_TILES = 16
_HPT = _NP // _TILES
_LAST = _N - 15 * _HPT
_G = _HPT // 16
_BN = 2048
_GRID = _NP // _BN
_NBIG = 1 << 30


def _sc_body(tid_hbm, beta_hbm, xs_hbm, ys_hbm,
             bmax_o, cnt_o, ax_o, ay_o, axn_o, ayn_o, bmn_o, cnn_o,
             tidv, betav, tbl_b, tbl_c, tbl_a,
             g_b, g_c, g_a, aclip, g_ax, g_ay,
             stage, stagei, axnv, aynv, bmnv, cnnv,
             sh_b, sh_c, sh_a, sem):
    wid = lax.axis_index("s")
    base = wid * _HPT
    lane = lax.iota(jnp.int32, 16)
    lane_off = lane * _K

    @pl.when(wid < _TILES - 1)
    def _full_load():
        pltpu.sync_copy(tid_hbm.at[pl.ds(base, _HPT)], tidv)
        pltpu.sync_copy(beta_hbm.at[pl.ds(base, _HPT)], betav)

    @pl.when(wid == _TILES - 1)
    def _tail_load():
        pltpu.sync_copy(tid_hbm.at[pl.ds(base, _LAST)],
                        tidv.at[pl.ds(0, _LAST)])
        pltpu.sync_copy(beta_hbm.at[pl.ds(base, _LAST)],
                        betav.at[pl.ds(0, _LAST)])

        def fill(j, c):
            s = _LAST + j * 16
            tidv[pl.ds(s, 16)] = jnp.zeros((16,), jnp.int32)
            betav[pl.ds(s, 16)] = jnp.zeros((16,), jnp.float32)
            return c
        lax.fori_loop(0, (_HPT - _LAST) // 16, fill, 0)

    def init_body(j, c):
        s = j * 16
        tbl_b[pl.ds(s, 16)] = jnp.full((16,), -1.0, jnp.float32)
        tbl_c[pl.ds(s, 16)] = jnp.zeros((16,), jnp.float32)
        tbl_a[pl.ds(s, 16)] = jnp.full((16,), _NBIG, jnp.int32)
        return c
    lax.fori_loop(0, _K, init_body, 0)

    def pass_a(i, c):
        off = i * 16
        tv = tidv[pl.ds(off, 16)]
        bv = betav[pl.ds(off, 16)]
        idx = lane_off + tv
        cb = plsc.load_gather(tbl_b, [idx])
        plsc.store_scatter(tbl_b, [idx], jnp.maximum(cb, bv))
        cc = plsc.load_gather(tbl_c, [idx])
        plsc.store_scatter(tbl_c, [idx], cc + 1.0)
        return c
    lax.fori_loop(0, _G, pass_a, 0)

    def lmerge_bc(j, c):
        s = j * 16
        accb = tbl_b[pl.ds(s, 16)]
        accc = tbl_c[pl.ds(s, 16)]
        for t in range(1, 16):
            accb = jnp.maximum(accb, tbl_b[pl.ds(t * _K + s, 16)])
            accc = accc + tbl_c[pl.ds(t * _K + s, 16)]
        g_b[pl.ds(s, 16)] = accb
        g_c[pl.ds(s, 16)] = accc
        return c
    lax.fori_loop(0, 16, lmerge_bc, 0)

    pltpu.sync_copy(g_b, sh_b.at[pl.ds(wid * _K, _K)])
    pltpu.sync_copy(g_c, sh_c.at[pl.ds(wid * _K, _K)])
    plsc.subcore_barrier()

    pltpu.sync_copy(sh_b, stage)

    def gmerge_b(j, c):
        s = j * 16
        acc = stage[pl.ds(s, 16)]
        for t in range(1, _TILES):
            acc = jnp.maximum(acc, stage[pl.ds(t * _K + s, 16)])
        g_b[pl.ds(s, 16)] = acc
        return c
    lax.fori_loop(0, 16, gmerge_b, 0)

    pltpu.sync_copy(sh_c, stage)

    def gmerge_c(j, c):
        s = j * 16
        acc = stage[pl.ds(s, 16)]
        for t in range(1, _TILES):
            acc = acc + stage[pl.ds(t * _K + s, 16)]
        g_c[pl.ds(s, 16)] = acc
        return c
    lax.fori_loop(0, 16, gmerge_c, 0)

    def pass_b1(i, c):
        off = i * 16
        tv = tidv[pl.ds(off, 16)]
        bv = betav[pl.ds(off, 16)]
        gb = plsc.load_gather(g_b, [tv])
        gidx = base + off + lane
        idx = lane_off + tv
        cur = plsc.load_gather(tbl_a, [idx])
        new = jnp.where(bv == gb, jnp.minimum(cur, gidx), cur)
        plsc.store_scatter(tbl_a, [idx], new)
        return c
    lax.fori_loop(0, _G, pass_b1, 0)

    def lmerge_a(j, c):
        s = j * 16
        acc = tbl_a[pl.ds(s, 16)]
        for t in range(1, 16):
            acc = jnp.minimum(acc, tbl_a[pl.ds(t * _K + s, 16)])
        g_a[pl.ds(s, 16)] = acc
        return c
    lax.fori_loop(0, 16, lmerge_a, 0)

    pltpu.sync_copy(g_a, sh_a.at[pl.ds(wid * _K, _K)])
    plsc.subcore_barrier()
    pltpu.sync_copy(sh_a, stagei)

    def gmerge_a(j, c):
        s = j * 16
        acc = stagei[pl.ds(s, 16)]
        for t in range(1, _TILES):
            acc = jnp.minimum(acc, stagei[pl.ds(t * _K + s, 16)])
        g_a[pl.ds(s, 16)] = acc
        aclip[pl.ds(s, 16)] = jnp.minimum(acc, _N - 1)
        return c
    lax.fori_loop(0, 16, gmerge_a, 0)

    pltpu.async_copy(xs_hbm.at[aclip], g_ax, sem).wait()
    pltpu.async_copy(ys_hbm.at[aclip], g_ay, sem).wait()

    def pass_b2(i, c):
        off = i * 16
        tv = tidv[pl.ds(off, 16)]
        axnv[pl.ds(off, 16)] = plsc.load_gather(g_ax, [tv])
        aynv[pl.ds(off, 16)] = plsc.load_gather(g_ay, [tv])
        bmnv[pl.ds(off, 16)] = plsc.load_gather(g_b, [tv])
        cnnv[pl.ds(off, 16)] = plsc.load_gather(g_c, [tv])
        return c
    lax.fori_loop(0, _G, pass_b2, 0)

    pltpu.sync_copy(axnv, axn_o.at[pl.ds(base, _HPT)])
    pltpu.sync_copy(aynv, ayn_o.at[pl.ds(base, _HPT)])
    pltpu.sync_copy(bmnv, bmn_o.at[pl.ds(base, _HPT)])
    pltpu.sync_copy(cnnv, cnn_o.at[pl.ds(base, _HPT)])

    @pl.when(wid == 0)
    def _emit_tables():
        pltpu.sync_copy(g_b, bmax_o)
        pltpu.sync_copy(g_c, cnt_o)
        pltpu.sync_copy(g_ax, ax_o)
        pltpu.sync_copy(g_ay, ay_o)


def _sc_call(tid_raw, beta_raw, xs, ys):
    f32 = jnp.float32
    kern = pl.kernel(
        _sc_body,
        out_type=[
            jax.ShapeDtypeStruct((_K,), f32),
            jax.ShapeDtypeStruct((_K,), f32),
            jax.ShapeDtypeStruct((_K,), f32),
            jax.ShapeDtypeStruct((_K,), f32),
            jax.ShapeDtypeStruct((_NP,), f32),
            jax.ShapeDtypeStruct((_NP,), f32),
            jax.ShapeDtypeStruct((_NP,), f32),
            jax.ShapeDtypeStruct((_NP,), f32),
        ],
        mesh=plsc.VectorSubcoreMesh(
            core_axis_name="c", subcore_axis_name="s",
            num_cores=1, num_subcores=_TILES),
        compiler_params=pltpu.CompilerParams(needs_layout_passes=False),
        scratch_types=[
            pltpu.VMEM((_HPT,), jnp.int32),
            pltpu.VMEM((_HPT,), f32),
            pltpu.VMEM((16 * _K,), f32),
            pltpu.VMEM((16 * _K,), f32),
            pltpu.VMEM((16 * _K,), jnp.int32),
            pltpu.VMEM((_K,), f32),
            pltpu.VMEM((_K,), f32),
            pltpu.VMEM((_K,), jnp.int32),
            pltpu.VMEM((_K,), jnp.int32),
            pltpu.VMEM((_K,), f32),
            pltpu.VMEM((_K,), f32),
            pltpu.VMEM((_TILES * _K,), f32),
            pltpu.VMEM((_TILES * _K,), jnp.int32),
            pltpu.VMEM((_HPT,), f32),
            pltpu.VMEM((_HPT,), f32),
            pltpu.VMEM((_HPT,), f32),
            pltpu.VMEM((_HPT,), f32),
            pltpu.VMEM_SHARED((_TILES * _K,), f32),
            pltpu.VMEM_SHARED((_TILES * _K,), f32),
            pltpu.VMEM_SHARED((_TILES * _K,), jnp.int32),
            pltpu.SemaphoreType.DMA,
        ],
    )
    return kern(tid_raw, beta_raw, xs, ys)


def _tc_body(cc_ref, beta_ref, tid_ref, pe_ref, te_ref, xs_ref, ys_ref,
             axn_ref, ayn_ref, bmn_ref, cnn_ref,
             ax_ref, ay_ref, bmax_ref, cntk_ref,
             out_ref, rep_ref, scal_ref):
    i = pl.program_id(0)

    @pl.when(i == 0)
    def _init():
        rep_ref[...] = jnp.zeros_like(rep_ref)
        scal_ref[...] = jnp.zeros_like(scal_ref)

    gidx = i * _BN + lax.broadcasted_iota(jnp.int32, (1, _BN), 1)
    real = gidx < _N
    beta = jnp.clip(beta_ref[...], 0.0, 1.0 - 1e-4)
    ath = 0.5 * jnp.log((1.0 + beta) / (1.0 - beta))
    q = jnp.where(real, ath * ath + _QMIN, 0.0)

    xs_col = cc_ref[:, 0:1]
    ys_col = cc_ref[:, 1:2]
    dx = xs_col - ax_ref[...]
    dy = ys_col - ay_ref[...]
    dist = jnp.sqrt(dx * dx + dy * dy + 1e-6)
    hinge = jnp.maximum(0.0, 1.0 - dist)
    rep_ref[...] += jax.lax.dot_general(
        q, hinge, (((1,), (0,)), ((), ())),
        preferred_element_type=jnp.float32)

    tid = tid_ref[...]
    cnn = cnn_ref[...]
    ownb = real & (tid > 0) & (cnn > 0.0)
    bmn = jnp.clip(bmn_ref[...], 0.0, 1.0 - 1e-4)
    athn = 0.5 * jnp.log((1.0 + bmn) / (1.0 - bmn))
    qan = athn * athn + _QMIN
    dxo = xs_ref[...] - axn_ref[...]
    dyo = ys_ref[...] - ayn_ref[...]
    dsq_own = dxo * dxo + dyo * dyo
    hinge_own = jnp.maximum(0.0, 1.0 - jnp.sqrt(dsq_own + 1e-6))
    qq = q * qan
    att_s = jnp.sum(jnp.where(ownb, qq * dsq_own / (cnn + 1e-3), 0.0))
    rep_own_s = jnp.sum(jnp.where(ownb, qq * hinge_own, 0.0))

    is_noise = (tid == 0) & real
    not_noise = (tid != 0) & real
    pe = pe_ref[...]
    te = te_ref[...]
    eph = (pe - te) ** 2 / (te + 1.0)
    s0 = jnp.sum(jnp.where(is_noise, beta, 0.0))
    s1 = jnp.sum(is_noise.astype(jnp.float32))
    s2 = jnp.sum(jnp.where(not_noise, eph, 0.0))

    lanev = lax.broadcasted_iota(jnp.int32, (1, 128), 1)
    upd = jnp.where(lanev == 0, att_s,
          jnp.where(lanev == 1, rep_own_s,
          jnp.where(lanev == 2, s0,
          jnp.where(lanev == 3, s1,
          jnp.where(lanev == 4, s2, 0.0)))))
    scal_ref[...] += upd

    @pl.when(i == _GRID - 1)
    def _fin():
        kio = lax.broadcasted_iota(jnp.int32, (1, _K), 1)
        cntk = cntk_ref[...]
        validk = ((kio > 0) & (cntk > 0.0)).astype(jnp.float32)
        bmk = jnp.clip(bmax_ref[...], 0.0, 1.0 - 1e-4)
        athk = 0.5 * jnp.log((1.0 + bmk) / (1.0 - bmk))
        w_rep = (athk * athk + _QMIN) * validk
        n_obj = jnp.sum(validk) + 1e-3
        att_t = scal_ref[0, 0]
        rep_own_t = scal_ref[0, 1]
        t0 = scal_ref[0, 2]
        t1 = scal_ref[0, 3]
        t2 = scal_ref[0, 4]
        v_att = att_t / n_obj
        rep_total = jnp.sum(rep_ref[...] * w_rep) - rep_own_t
        v_rep = rep_total / (float(_N) * n_obj)
        beta_pen = jnp.sum((1.0 - bmax_ref[...]) * validk) / n_obj
        noise_pen = _SB * t0 / (t1 + 1e-3)
        eloss = t2 / ((float(_N) - t1) + 1e-3)
        loss = v_att + v_rep + beta_pen + noise_pen + eloss
        lv = lax.broadcasted_iota(jnp.int32, (1, 128), 1)
        out_ref[...] = jnp.where(lv == 0, loss, 0.0)


def _tc_call(cc_pad, beta_r, tid_r, pe_r, te_r, xs_r, ys_r,
             axn_r, ayn_r, bmn_r, cnn_r, ax2, ay2, bmax2, cnt2):
    row_n = pl.BlockSpec((1, _BN), lambda i: (0, i))
    row_p = pl.BlockSpec((1, _BN), lambda i: (0, i))
    kc = pl.BlockSpec((1, _K), lambda i: (0, 0))
    return pl.pallas_call(
        _tc_body,
        grid=(_GRID,),
        in_specs=[pl.BlockSpec((_BN, 2), lambda i: (i, 0)),
                  row_n, row_n, row_n, row_n, row_n, row_n,
                  row_p, row_p, row_p, row_p,
                  kc, kc, kc, kc],
        out_specs=pl.BlockSpec((1, 128), lambda i: (0, 0)),
        out_shape=jax.ShapeDtypeStruct((1, 128), jnp.float32),
        scratch_shapes=[
            pltpu.VMEM((1, _K), jnp.float32),
            pltpu.VMEM((1, 128), jnp.float32),
        ],
    )(cc_pad, beta_r, tid_r, pe_r, te_r, xs_r, ys_r,
      axn_r, ayn_r, bmn_r, cnn_r, ax2, ay2, bmax2, cnt2)


def kernel(pred_beta, pred_ccoords, pred_energy, t_idx, t_energy):
    tid_raw = t_idx[:, 0]
    beta_raw = pred_beta[:, 0]
    xs = pred_ccoords[:, 0]
    ys = pred_ccoords[:, 1]

    bmax, counts, ax, ay, axn, ayn, bmn, cnn = _sc_call(
        tid_raw, beta_raw, xs, ys)

    cc_pad = jnp.pad(pred_ccoords, ((0, _NP - _N), (0, 0)))

    out = _tc_call(
        cc_pad,
        beta_raw[None, :], tid_raw[None, :],
        pred_energy[:, 0][None, :], t_energy[:, 0][None, :],
        xs[None, :], ys[None, :],
        axn[None, :], ayn[None, :], bmn[None, :], cnn[None, :],
        ax[None, :], ay[None, :], bmax[None, :], counts[None, :])
    return out[0, 0]


# speedup vs baseline: 2.6926x; 1.1349x over previous
import jax
import jax.numpy as jnp
from jax import lax
from jax.experimental import pallas as pl
from jax.experimental.pallas import tpu as pltpu
from jax.experimental.pallas import tpu_sc as plsc

_N = 100000
_K = 256
_QMIN = 0.1
_SB = 1.0

_NP = 100352
_TILES = 16
_HPT = _NP // _TILES
_LAST = _N - 15 * _HPT
_G = _HPT // 16
_BN = 2048
_GRID = _NP // _BN
_NBIG = 1 << 30


def _sc_body(tid_hbm, beta_hbm, xs_hbm, ys_hbm,
             bmax_o, cnt_o, ax_o, ay_o, axn_o, ayn_o, bmn_o, cnn_o,
             tidv, betav, tbl_b, tbl_c, tbl_a,
             g_b, g_c, g_a, aclip, g_ax, g_ay,
             stage, stagei, axnv, aynv, bmnv, cnnv,
             sh_b, sh_c, sh_a, sem):
    wid = lax.axis_index("s")
    base = wid * _HPT
    lane = lax.iota(jnp.int32, 16)
    lane_off = lane * _K

    @pl.when(wid < _TILES - 1)
    def _full_load():
        pltpu.sync_copy(tid_hbm.at[pl.ds(base, _HPT)], tidv)
        pltpu.sync_copy(beta_hbm.at[pl.ds(base, _HPT)], betav)

    @pl.when(wid == _TILES - 1)
    def _tail_load():
        pltpu.sync_copy(tid_hbm.at[pl.ds(base, _LAST)],
                        tidv.at[pl.ds(0, _LAST)])
        pltpu.sync_copy(beta_hbm.at[pl.ds(base, _LAST)],
                        betav.at[pl.ds(0, _LAST)])

        def fill(j, c):
            s = _LAST + j * 16
            tidv[pl.ds(s, 16)] = jnp.zeros((16,), jnp.int32)
            betav[pl.ds(s, 16)] = jnp.zeros((16,), jnp.float32)
            return c
        lax.fori_loop(0, (_HPT - _LAST) // 16, fill, 0)

    def init_body(j, c):
        s = j * 16
        tbl_b[pl.ds(s, 16)] = jnp.full((16,), -1.0, jnp.float32)
        tbl_c[pl.ds(s, 16)] = jnp.zeros((16,), jnp.float32)
        tbl_a[pl.ds(s, 16)] = jnp.full((16,), _NBIG, jnp.int32)
        return c
    lax.fori_loop(0, _K, init_body, 0)

    def pass_a(i, c):
        off = i * 16
        tv = tidv[pl.ds(off, 16)]
        bv = betav[pl.ds(off, 16)]
        idx = lane_off + tv
        cb = plsc.load_gather(tbl_b, [idx])
        plsc.store_scatter(tbl_b, [idx], jnp.maximum(cb, bv))
        cc = plsc.load_gather(tbl_c, [idx])
        plsc.store_scatter(tbl_c, [idx], cc + 1.0)
        return c
    lax.fori_loop(0, _G, pass_a, 0)

    def lmerge_bc(j, c):
        s = j * 16
        accb = tbl_b[pl.ds(s, 16)]
        accc = tbl_c[pl.ds(s, 16)]
        for t in range(1, 16):
            accb = jnp.maximum(accb, tbl_b[pl.ds(t * _K + s, 16)])
            accc = accc + tbl_c[pl.ds(t * _K + s, 16)]
        g_b[pl.ds(s, 16)] = accb
        g_c[pl.ds(s, 16)] = accc
        return c
    lax.fori_loop(0, 16, lmerge_bc, 0)

    pltpu.sync_copy(g_b, sh_b.at[pl.ds(wid * _K, _K)])
    pltpu.sync_copy(g_c, sh_c.at[pl.ds(wid * _K, _K)])
    plsc.subcore_barrier()

    pltpu.sync_copy(sh_b, stage)

    def gmerge_b(j, c):
        s = j * 16
        acc = stage[pl.ds(s, 16)]
        for t in range(1, _TILES):
            acc = jnp.maximum(acc, stage[pl.ds(t * _K + s, 16)])
        g_b[pl.ds(s, 16)] = acc
        return c
    lax.fori_loop(0, 16, gmerge_b, 0)

    pltpu.sync_copy(sh_c, stage)

    def gmerge_c(j, c):
        s = j * 16
        acc = stage[pl.ds(s, 16)]
        for t in range(1, _TILES):
            acc = acc + stage[pl.ds(t * _K + s, 16)]
        g_c[pl.ds(s, 16)] = acc
        return c
    lax.fori_loop(0, 16, gmerge_c, 0)

    def pass_b1(i, c):
        off = i * 16
        tv = tidv[pl.ds(off, 16)]
        bv = betav[pl.ds(off, 16)]
        gb = plsc.load_gather(g_b, [tv])
        gidx = base + off + lane
        idx = lane_off + tv
        cur = plsc.load_gather(tbl_a, [idx])
        new = jnp.where(bv == gb, jnp.minimum(cur, gidx), cur)
        plsc.store_scatter(tbl_a, [idx], new)
        return c
    lax.fori_loop(0, _G, pass_b1, 0)

    def lmerge_a(j, c):
        s = j * 16
        acc = tbl_a[pl.ds(s, 16)]
        for t in range(1, 16):
            acc = jnp.minimum(acc, tbl_a[pl.ds(t * _K + s, 16)])
        g_a[pl.ds(s, 16)] = acc
        return c
    lax.fori_loop(0, 16, lmerge_a, 0)

    pltpu.sync_copy(g_a, sh_a.at[pl.ds(wid * _K, _K)])
    plsc.subcore_barrier()
    pltpu.sync_copy(sh_a, stagei)

    def gmerge_a(j, c):
        s = j * 16
        acc = stagei[pl.ds(s, 16)]
        for t in range(1, _TILES):
            acc = jnp.minimum(acc, stagei[pl.ds(t * _K + s, 16)])
        g_a[pl.ds(s, 16)] = acc
        aclip[pl.ds(s, 16)] = jnp.minimum(acc, _N - 1)
        return c
    lax.fori_loop(0, 16, gmerge_a, 0)

    pltpu.async_copy(xs_hbm.at[aclip], g_ax, sem).wait()
    pltpu.async_copy(ys_hbm.at[aclip], g_ay, sem).wait()

    def pass_b2(i, c):
        off = i * 16
        tv = tidv[pl.ds(off, 16)]
        axnv[pl.ds(off, 16)] = plsc.load_gather(g_ax, [tv])
        aynv[pl.ds(off, 16)] = plsc.load_gather(g_ay, [tv])
        bmnv[pl.ds(off, 16)] = plsc.load_gather(g_b, [tv])
        cnnv[pl.ds(off, 16)] = plsc.load_gather(g_c, [tv])
        return c
    lax.fori_loop(0, _G, pass_b2, 0)

    pltpu.sync_copy(axnv, axn_o.at[pl.ds(base, _HPT)])
    pltpu.sync_copy(aynv, ayn_o.at[pl.ds(base, _HPT)])
    pltpu.sync_copy(bmnv, bmn_o.at[pl.ds(base, _HPT)])
    pltpu.sync_copy(cnnv, cnn_o.at[pl.ds(base, _HPT)])

    @pl.when(wid == 0)
    def _emit_tables():
        pltpu.sync_copy(g_b, bmax_o)
        pltpu.sync_copy(g_c, cnt_o)
        pltpu.sync_copy(g_ax, ax_o)
        pltpu.sync_copy(g_ay, ay_o)


def _sc_call(tid_raw, beta_raw, xs, ys):
    f32 = jnp.float32
    kern = pl.kernel(
        _sc_body,
        out_type=[
            jax.ShapeDtypeStruct((_K,), f32),
            jax.ShapeDtypeStruct((_K,), f32),
            jax.ShapeDtypeStruct((_K,), f32),
            jax.ShapeDtypeStruct((_K,), f32),
            jax.ShapeDtypeStruct((_NP,), f32),
            jax.ShapeDtypeStruct((_NP,), f32),
            jax.ShapeDtypeStruct((_NP,), f32),
            jax.ShapeDtypeStruct((_NP,), f32),
        ],
        mesh=plsc.VectorSubcoreMesh(
            core_axis_name="c", subcore_axis_name="s",
            num_cores=1, num_subcores=_TILES),
        compiler_params=pltpu.CompilerParams(needs_layout_passes=False),
        scratch_types=[
            pltpu.VMEM((_HPT,), jnp.int32),
            pltpu.VMEM((_HPT,), f32),
            pltpu.VMEM((16 * _K,), f32),
            pltpu.VMEM((16 * _K,), f32),
            pltpu.VMEM((16 * _K,), jnp.int32),
            pltpu.VMEM((_K,), f32),
            pltpu.VMEM((_K,), f32),
            pltpu.VMEM((_K,), jnp.int32),
            pltpu.VMEM((_K,), jnp.int32),
            pltpu.VMEM((_K,), f32),
            pltpu.VMEM((_K,), f32),
            pltpu.VMEM((_TILES * _K,), f32),
            pltpu.VMEM((_TILES * _K,), jnp.int32),
            pltpu.VMEM((_HPT,), f32),
            pltpu.VMEM((_HPT,), f32),
            pltpu.VMEM((_HPT,), f32),
            pltpu.VMEM((_HPT,), f32),
            pltpu.VMEM_SHARED((_TILES * _K,), f32),
            pltpu.VMEM_SHARED((_TILES * _K,), f32),
            pltpu.VMEM_SHARED((_TILES * _K,), jnp.int32),
            pltpu.SemaphoreType.DMA,
        ],
    )
    return kern(tid_raw, beta_raw, xs, ys)


def _tc_body(cc_ref, beta_ref, tid_ref, pe_ref, te_ref, xs_ref, ys_ref,
             axn_ref, ayn_ref, bmn_ref, cnn_ref,
             ax_ref, ay_ref, bmax_ref, cntk_ref,
             out_ref, rep_ref, scal_ref):
    i = pl.program_id(0)

    @pl.when(i == 0)
    def _init():
        rep_ref[...] = jnp.zeros_like(rep_ref)
        scal_ref[...] = jnp.zeros_like(scal_ref)

    gidx = i * _BN + lax.broadcasted_iota(jnp.int32, (1, _BN), 1)
    real = gidx < _N
    beta = jnp.clip(beta_ref[...], 0.0, 1.0 - 1e-4)
    ath = 0.5 * jnp.log((1.0 + beta) / (1.0 - beta))
    q = jnp.where(real, ath * ath + _QMIN, 0.0)

    real_col = (i * _BN + lax.broadcasted_iota(jnp.int32, (_BN, 1), 0)) < _N
    xs_col = cc_ref[:, 0:1]
    ys_col = cc_ref[:, 1:2]
    dx = xs_col - ax_ref[...]
    dy = ys_col - ay_ref[...]
    dist = jnp.sqrt(dx * dx + dy * dy + 1e-6)
    hinge = jnp.where(real_col, jnp.maximum(0.0, 1.0 - dist), 0.0)
    rep_ref[...] += jax.lax.dot_general(
        q, hinge, (((1,), (0,)), ((), ())),
        preferred_element_type=jnp.float32)

    tid = tid_ref[...]
    cnn = cnn_ref[...]
    ownb = real & (tid > 0) & (cnn > 0.0)
    bmn = jnp.clip(bmn_ref[...], 0.0, 1.0 - 1e-4)
    athn = 0.5 * jnp.log((1.0 + bmn) / (1.0 - bmn))
    qan = athn * athn + _QMIN
    dxo = xs_ref[...] - axn_ref[...]
    dyo = ys_ref[...] - ayn_ref[...]
    dsq_own = dxo * dxo + dyo * dyo
    hinge_own = jnp.maximum(0.0, 1.0 - jnp.sqrt(dsq_own + 1e-6))
    qq = q * qan
    att_s = jnp.sum(jnp.where(ownb, qq * dsq_own / (cnn + 1e-3), 0.0))
    rep_own_s = jnp.sum(jnp.where(ownb, qq * hinge_own, 0.0))

    is_noise = (tid == 0) & real
    not_noise = (tid != 0) & real
    pe = pe_ref[...]
    te = te_ref[...]
    eph = (pe - te) ** 2 / (te + 1.0)
    s0 = jnp.sum(jnp.where(is_noise, beta, 0.0))
    s1 = jnp.sum(is_noise.astype(jnp.float32))
    s2 = jnp.sum(jnp.where(not_noise, eph, 0.0))

    lanev = lax.broadcasted_iota(jnp.int32, (1, 128), 1)
    upd = jnp.where(lanev == 0, att_s,
          jnp.where(lanev == 1, rep_own_s,
          jnp.where(lanev == 2, s0,
          jnp.where(lanev == 3, s1,
          jnp.where(lanev == 4, s2, 0.0)))))
    scal_ref[...] += upd

    @pl.when(i == _GRID - 1)
    def _fin():
        kio = lax.broadcasted_iota(jnp.int32, (1, _K), 1)
        cntk = cntk_ref[...]
        validk = ((kio > 0) & (cntk > 0.0)).astype(jnp.float32)
        bmk = jnp.clip(bmax_ref[...], 0.0, 1.0 - 1e-4)
        athk = 0.5 * jnp.log((1.0 + bmk) / (1.0 - bmk))
        w_rep = (athk * athk + _QMIN) * validk
        n_obj = jnp.sum(validk) + 1e-3
        att_t = scal_ref[0, 0]
        rep_own_t = scal_ref[0, 1]
        t0 = scal_ref[0, 2]
        t1 = scal_ref[0, 3]
        t2 = scal_ref[0, 4]
        v_att = att_t / n_obj
        rep_total = jnp.sum(rep_ref[...] * w_rep) - rep_own_t
        v_rep = rep_total / (float(_N) * n_obj)
        beta_pen = jnp.sum((1.0 - bmax_ref[...]) * validk) / n_obj
        noise_pen = _SB * t0 / (t1 + 1e-3)
        eloss = t2 / ((float(_N) - t1) + 1e-3)
        loss = v_att + v_rep + beta_pen + noise_pen + eloss
        lv = lax.broadcasted_iota(jnp.int32, (1, 128), 1)
        out_ref[...] = jnp.where(lv == 0, loss, 0.0)


def _tc_call(cc, beta_r, tid_r, pe_r, te_r, xs_r, ys_r,
             axn_r, ayn_r, bmn_r, cnn_r, ax2, ay2, bmax2, cnt2):
    row_n = pl.BlockSpec((1, _BN), lambda i: (0, i))
    row_p = pl.BlockSpec((1, _BN), lambda i: (0, i))
    kc = pl.BlockSpec((1, _K), lambda i: (0, 0))
    return pl.pallas_call(
        _tc_body,
        grid=(_GRID,),
        in_specs=[pl.BlockSpec((_BN, 2), lambda i: (i, 0)),
                  row_n, row_n, row_n, row_n, row_n, row_n,
                  row_p, row_p, row_p, row_p,
                  kc, kc, kc, kc],
        out_specs=pl.BlockSpec((1, 128), lambda i: (0, 0)),
        out_shape=jax.ShapeDtypeStruct((1, 128), jnp.float32),
        scratch_shapes=[
            pltpu.VMEM((1, _K), jnp.float32),
            pltpu.VMEM((1, 128), jnp.float32),
        ],
    )(cc, beta_r, tid_r, pe_r, te_r, xs_r, ys_r,
      axn_r, ayn_r, bmn_r, cnn_r, ax2, ay2, bmax2, cnt2)


def kernel(pred_beta, pred_ccoords, pred_energy, t_idx, t_energy):
    tid_raw = t_idx[:, 0]
    beta_raw = pred_beta[:, 0]
    xs = pred_ccoords[:, 0]
    ys = pred_ccoords[:, 1]

    bmax, counts, ax, ay, axn, ayn, bmn, cnn = _sc_call(
        tid_raw, beta_raw, xs, ys)

    out = _tc_call(
        pred_ccoords,
        beta_raw[None, :], tid_raw[None, :],
        pred_energy[:, 0][None, :], t_energy[:, 0][None, :],
        xs[None, :], ys[None, :],
        axn[None, :], ayn[None, :], bmn[None, :], cnn[None, :],
        ax[None, :], ay[None, :], bmax[None, :], counts[None, :])
    return out[0, 0]


# speedup vs baseline: 3.2555x; 1.2090x over previous
import jax
import jax.numpy as jnp
from jax import lax
from jax.experimental import pallas as pl
from jax.experimental.pallas import tpu as pltpu
from jax.experimental.pallas import tpu_sc as plsc

_N = 100000
_K = 256
_QMIN = 0.1
_SB = 1.0

_NP = 100352
_TILES = 16
_HPT = _NP // _TILES
_LAST = _N - 15 * _HPT
_G = _HPT // 16
_BN = 2048
_GRID = _NP // _BN
_NBIG = 1 << 30


def _sc_body(tid_hbm, beta_hbm, xs_hbm, ys_hbm,
             bmax_o, cnt_o, ax_o, ay_o, axn_o, ayn_o, bmn_o, cnn_o,
             tidv, betav, tbl_b, tbl_c, tbl_a,
             g_b, g_c, g_a, aclip, g_ax, g_ay,
             stage, stagei, axnv, aynv, bmnv, cnnv,
             sh_b, sh_c, sh_a, sem):
    wid = lax.axis_index("s")
    base = wid * _HPT
    lane = lax.iota(jnp.int32, 16)
    lane_off = lane * _K

    @pl.when(wid < _TILES - 1)
    def _full_load():
        pltpu.sync_copy(tid_hbm.at[pl.ds(base, _HPT)], tidv)
        pltpu.sync_copy(beta_hbm.at[pl.ds(base, _HPT)], betav)

    @pl.when(wid == _TILES - 1)
    def _tail_load():
        pltpu.sync_copy(tid_hbm.at[pl.ds(base, _LAST)],
                        tidv.at[pl.ds(0, _LAST)])
        pltpu.sync_copy(beta_hbm.at[pl.ds(base, _LAST)],
                        betav.at[pl.ds(0, _LAST)])

        def fill(j, c):
            s = _LAST + j * 16
            tidv[pl.ds(s, 16)] = jnp.zeros((16,), jnp.int32)
            betav[pl.ds(s, 16)] = jnp.zeros((16,), jnp.float32)
            return c
        lax.fori_loop(0, (_HPT - _LAST) // 16, fill, 0)

    def init_body(j, c):
        s = j * 16
        tbl_b[pl.ds(s, 16)] = jnp.full((16,), -1.0, jnp.float32)
        tbl_c[pl.ds(s, 16)] = jnp.zeros((16,), jnp.float32)
        tbl_a[pl.ds(s, 16)] = jnp.full((16,), _NBIG, jnp.int32)
        return c
    lax.fori_loop(0, _K, init_body, 0)

    def pass_a(i, c):
        off = i * 16
        tv = tidv[pl.ds(off, 16)]
        bv = betav[pl.ds(off, 16)]
        idx = lane_off + tv
        cb = plsc.load_gather(tbl_b, [idx])
        plsc.store_scatter(tbl_b, [idx], jnp.maximum(cb, bv))
        cc = plsc.load_gather(tbl_c, [idx])
        plsc.store_scatter(tbl_c, [idx], cc + 1.0)
        return c
    lax.fori_loop(0, _G, pass_a, 0)

    def lmerge_bc(j, c):
        s = j * 16
        accb = tbl_b[pl.ds(s, 16)]
        accc = tbl_c[pl.ds(s, 16)]
        for t in range(1, 16):
            accb = jnp.maximum(accb, tbl_b[pl.ds(t * _K + s, 16)])
            accc = accc + tbl_c[pl.ds(t * _K + s, 16)]
        g_b[pl.ds(s, 16)] = accb
        g_c[pl.ds(s, 16)] = accc
        return c
    lax.fori_loop(0, 16, lmerge_bc, 0)

    pltpu.sync_copy(g_b, sh_b.at[pl.ds(wid * _K, _K)])
    pltpu.sync_copy(g_c, sh_c.at[pl.ds(wid * _K, _K)])
    plsc.subcore_barrier()

    pltpu.sync_copy(sh_b, stage)

    def gmerge_b(j, c):
        s = j * 16
        acc = stage[pl.ds(s, 16)]
        for t in range(1, _TILES):
            acc = jnp.maximum(acc, stage[pl.ds(t * _K + s, 16)])
        g_b[pl.ds(s, 16)] = acc
        return c
    lax.fori_loop(0, 16, gmerge_b, 0)

    pltpu.sync_copy(sh_c, stage)

    def gmerge_c(j, c):
        s = j * 16
        acc = stage[pl.ds(s, 16)]
        for t in range(1, _TILES):
            acc = acc + stage[pl.ds(t * _K + s, 16)]
        g_c[pl.ds(s, 16)] = acc
        return c
    lax.fori_loop(0, 16, gmerge_c, 0)

    def pass_b1(i, c):
        off = i * 16
        tv = tidv[pl.ds(off, 16)]
        bv = betav[pl.ds(off, 16)]
        gb = plsc.load_gather(g_b, [tv])
        gidx = base + off + lane
        idx = lane_off + tv
        cur = plsc.load_gather(tbl_a, [idx])
        new = jnp.where(bv == gb, jnp.minimum(cur, gidx), cur)
        plsc.store_scatter(tbl_a, [idx], new)
        return c
    lax.fori_loop(0, _G, pass_b1, 0)

    def lmerge_a(j, c):
        s = j * 16
        acc = tbl_a[pl.ds(s, 16)]
        for t in range(1, 16):
            acc = jnp.minimum(acc, tbl_a[pl.ds(t * _K + s, 16)])
        g_a[pl.ds(s, 16)] = acc
        return c
    lax.fori_loop(0, 16, lmerge_a, 0)

    pltpu.sync_copy(g_a, sh_a.at[pl.ds(wid * _K, _K)])
    plsc.subcore_barrier()
    pltpu.sync_copy(sh_a, stagei)

    def gmerge_a(j, c):
        s = j * 16
        acc = stagei[pl.ds(s, 16)]
        for t in range(1, _TILES):
            acc = jnp.minimum(acc, stagei[pl.ds(t * _K + s, 16)])
        g_a[pl.ds(s, 16)] = acc
        aclip[pl.ds(s, 16)] = jnp.minimum(acc, _N - 1)
        return c
    lax.fori_loop(0, 16, gmerge_a, 0)

    pltpu.async_copy(xs_hbm.at[aclip], g_ax, sem).wait()
    pltpu.async_copy(ys_hbm.at[aclip], g_ay, sem).wait()

    def pass_b2(i, c):
        off = i * 16
        tv = tidv[pl.ds(off, 16)]
        axnv[pl.ds(off, 16)] = plsc.load_gather(g_ax, [tv])
        aynv[pl.ds(off, 16)] = plsc.load_gather(g_ay, [tv])
        bmnv[pl.ds(off, 16)] = plsc.load_gather(g_b, [tv])
        cnnv[pl.ds(off, 16)] = plsc.load_gather(g_c, [tv])
        return c
    lax.fori_loop(0, _G, pass_b2, 0)

    pltpu.sync_copy(axnv, axn_o.at[pl.ds(base, _HPT)])
    pltpu.sync_copy(aynv, ayn_o.at[pl.ds(base, _HPT)])
    pltpu.sync_copy(bmnv, bmn_o.at[pl.ds(base, _HPT)])
    pltpu.sync_copy(cnnv, cnn_o.at[pl.ds(base, _HPT)])

    @pl.when(wid == 0)
    def _emit_tables():
        pltpu.sync_copy(g_b, bmax_o)
        pltpu.sync_copy(g_c, cnt_o)
        pltpu.sync_copy(g_ax, ax_o)
        pltpu.sync_copy(g_ay, ay_o)


def _sc_call(tid_raw, beta_raw, xs, ys):
    f32 = jnp.float32
    kern = pl.kernel(
        _sc_body,
        out_type=[
            jax.ShapeDtypeStruct((_K,), f32),
            jax.ShapeDtypeStruct((_K,), f32),
            jax.ShapeDtypeStruct((_K,), f32),
            jax.ShapeDtypeStruct((_K,), f32),
            jax.ShapeDtypeStruct((_NP,), f32),
            jax.ShapeDtypeStruct((_NP,), f32),
            jax.ShapeDtypeStruct((_NP,), f32),
            jax.ShapeDtypeStruct((_NP,), f32),
        ],
        mesh=plsc.VectorSubcoreMesh(
            core_axis_name="c", subcore_axis_name="s",
            num_cores=1, num_subcores=_TILES),
        compiler_params=pltpu.CompilerParams(needs_layout_passes=False),
        scratch_types=[
            pltpu.VMEM((_HPT,), jnp.int32),
            pltpu.VMEM((_HPT,), f32),
            pltpu.VMEM((16 * _K,), f32),
            pltpu.VMEM((16 * _K,), f32),
            pltpu.VMEM((16 * _K,), jnp.int32),
            pltpu.VMEM((_K,), f32),
            pltpu.VMEM((_K,), f32),
            pltpu.VMEM((_K,), jnp.int32),
            pltpu.VMEM((_K,), jnp.int32),
            pltpu.VMEM((_K,), f32),
            pltpu.VMEM((_K,), f32),
            pltpu.VMEM((_TILES * _K,), f32),
            pltpu.VMEM((_TILES * _K,), jnp.int32),
            pltpu.VMEM((_HPT,), f32),
            pltpu.VMEM((_HPT,), f32),
            pltpu.VMEM((_HPT,), f32),
            pltpu.VMEM((_HPT,), f32),
            pltpu.VMEM_SHARED((_TILES * _K,), f32),
            pltpu.VMEM_SHARED((_TILES * _K,), f32),
            pltpu.VMEM_SHARED((_TILES * _K,), jnp.int32),
            pltpu.SemaphoreType.DMA,
        ],
    )
    return kern(tid_raw, beta_raw, xs, ys)


def _tc_body(cc_ref, beta_ref, tid_ref, pe_ref, te_ref, xs_ref, ys_ref,
             axn_ref, ayn_ref, bmn_ref, cnn_ref,
             ax_ref, ay_ref, bmax_ref, cntk_ref,
             out_ref, rep_ref, scal_ref):
    i = pl.program_id(0)

    @pl.when(i == 0)
    def _init():
        rep_ref[...] = jnp.zeros_like(rep_ref)
        scal_ref[...] = jnp.zeros_like(scal_ref)

    gidx = i * _BN + lax.broadcasted_iota(jnp.int32, (1, _BN), 1)
    real = gidx < _N
    beta = jnp.clip(beta_ref[...], 0.0, 1.0 - 1e-4)
    ath = 0.5 * jnp.log((1.0 + beta) / (1.0 - beta))
    q = jnp.where(real, ath * ath + _QMIN, 0.0)

    bf16 = jnp.bfloat16
    real_col = (i * _BN + lax.broadcasted_iota(jnp.int32, (_BN, 1), 0)) < _N
    xs_col = cc_ref[:, 0:1].astype(bf16)
    ys_col = cc_ref[:, 1:2].astype(bf16)
    dx = xs_col - ax_ref[...].astype(bf16)
    dy = ys_col - ay_ref[...].astype(bf16)
    dist = jnp.sqrt(dx * dx + dy * dy + bf16(1e-6))
    hinge = jnp.where(real_col, jnp.maximum(bf16(0.0), bf16(1.0) - dist),
                      bf16(0.0))
    rep_ref[...] += jax.lax.dot_general(
        q.astype(bf16), hinge, (((1,), (0,)), ((), ())),
        preferred_element_type=jnp.float32)

    tid = tid_ref[...]
    cnn = cnn_ref[...]
    ownb = real & (tid > 0) & (cnn > 0.0)
    bmn = jnp.clip(bmn_ref[...], 0.0, 1.0 - 1e-4)
    athn = 0.5 * jnp.log((1.0 + bmn) / (1.0 - bmn))
    qan = athn * athn + _QMIN
    dxo = xs_ref[...] - axn_ref[...]
    dyo = ys_ref[...] - ayn_ref[...]
    dsq_own = dxo * dxo + dyo * dyo
    hinge_own = jnp.maximum(0.0, 1.0 - jnp.sqrt(dsq_own + 1e-6))
    qq = q * qan
    att_s = jnp.sum(jnp.where(ownb, qq * dsq_own / (cnn + 1e-3), 0.0))
    rep_own_s = jnp.sum(jnp.where(ownb, qq * hinge_own, 0.0))

    is_noise = (tid == 0) & real
    not_noise = (tid != 0) & real
    pe = pe_ref[...]
    te = te_ref[...]
    eph = (pe - te) ** 2 / (te + 1.0)
    s0 = jnp.sum(jnp.where(is_noise, beta, 0.0))
    s1 = jnp.sum(is_noise.astype(jnp.float32))
    s2 = jnp.sum(jnp.where(not_noise, eph, 0.0))

    lanev = lax.broadcasted_iota(jnp.int32, (1, 128), 1)
    upd = jnp.where(lanev == 0, att_s,
          jnp.where(lanev == 1, rep_own_s,
          jnp.where(lanev == 2, s0,
          jnp.where(lanev == 3, s1,
          jnp.where(lanev == 4, s2, 0.0)))))
    scal_ref[...] += upd

    @pl.when(i == _GRID - 1)
    def _fin():
        kio = lax.broadcasted_iota(jnp.int32, (1, _K), 1)
        cntk = cntk_ref[...]
        validk = ((kio > 0) & (cntk > 0.0)).astype(jnp.float32)
        bmk = jnp.clip(bmax_ref[...], 0.0, 1.0 - 1e-4)
        athk = 0.5 * jnp.log((1.0 + bmk) / (1.0 - bmk))
        w_rep = (athk * athk + _QMIN) * validk
        n_obj = jnp.sum(validk) + 1e-3
        att_t = scal_ref[0, 0]
        rep_own_t = scal_ref[0, 1]
        t0 = scal_ref[0, 2]
        t1 = scal_ref[0, 3]
        t2 = scal_ref[0, 4]
        v_att = att_t / n_obj
        rep_total = jnp.sum(rep_ref[...] * w_rep) - rep_own_t
        v_rep = rep_total / (float(_N) * n_obj)
        beta_pen = jnp.sum((1.0 - bmax_ref[...]) * validk) / n_obj
        noise_pen = _SB * t0 / (t1 + 1e-3)
        eloss = t2 / ((float(_N) - t1) + 1e-3)
        loss = v_att + v_rep + beta_pen + noise_pen + eloss
        lv = lax.broadcasted_iota(jnp.int32, (1, 128), 1)
        out_ref[...] = jnp.where(lv == 0, loss, 0.0)


def _tc_call(cc, beta_r, tid_r, pe_r, te_r, xs_r, ys_r,
             axn_r, ayn_r, bmn_r, cnn_r, ax2, ay2, bmax2, cnt2):
    row_n = pl.BlockSpec((1, _BN), lambda i: (0, i))
    row_p = pl.BlockSpec((1, _BN), lambda i: (0, i))
    kc = pl.BlockSpec((1, _K), lambda i: (0, 0))
    return pl.pallas_call(
        _tc_body,
        grid=(_GRID,),
        in_specs=[pl.BlockSpec((_BN, 2), lambda i: (i, 0)),
                  row_n, row_n, row_n, row_n, row_n, row_n,
                  row_p, row_p, row_p, row_p,
                  kc, kc, kc, kc],
        out_specs=pl.BlockSpec((1, 128), lambda i: (0, 0)),
        out_shape=jax.ShapeDtypeStruct((1, 128), jnp.float32),
        scratch_shapes=[
            pltpu.VMEM((1, _K), jnp.float32),
            pltpu.VMEM((1, 128), jnp.float32),
        ],
    )(cc, beta_r, tid_r, pe_r, te_r, xs_r, ys_r,
      axn_r, ayn_r, bmn_r, cnn_r, ax2, ay2, bmax2, cnt2)


def kernel(pred_beta, pred_ccoords, pred_energy, t_idx, t_energy):
    tid_raw = t_idx[:, 0]
    beta_raw = pred_beta[:, 0]
    xs = pred_ccoords[:, 0]
    ys = pred_ccoords[:, 1]

    bmax, counts, ax, ay, axn, ayn, bmn, cnn = _sc_call(
        tid_raw, beta_raw, xs, ys)

    out = _tc_call(
        pred_ccoords,
        beta_raw[None, :], tid_raw[None, :],
        pred_energy[:, 0][None, :], t_energy[:, 0][None, :],
        xs[None, :], ys[None, :],
        axn[None, :], ayn[None, :], bmn[None, :], cnn[None, :],
        ax[None, :], ay[None, :], bmax[None, :], counts[None, :])
    return out[0, 0]


# speedup vs baseline: 3.4294x; 1.0534x over previous
import jax
import jax.numpy as jnp
from jax import lax
from jax.experimental import pallas as pl
from jax.experimental.pallas import tpu as pltpu
from jax.experimental.pallas import tpu_sc as plsc

_N = 100000
_K = 256
_QMIN = 0.1
_SB = 1.0

_NP = 100352
_TILES = 16
_HPT = _NP // _TILES
_LAST = _N - 15 * _HPT
_G = _HPT // 16
_BN = 3584
_GRID = _NP // _BN
_NBIG = 1 << 30


def _sc_body(tid_hbm, beta_hbm, xs_hbm, ys_hbm,
             bmax_o, cnt_o, ax_o, ay_o, axn_o, ayn_o, bmn_o, cnn_o,
             tidv, betav, tbl_b, tbl_c, tbl_a,
             g_b, g_c, g_a, aclip, g_ax, g_ay,
             stage, stagei, axnv, aynv, bmnv, cnnv,
             sh_b, sh_c, sh_a, sem):
    wid = lax.axis_index("s")
    base = wid * _HPT
    lane = lax.iota(jnp.int32, 16)
    lane_off = lane * _K

    @pl.when(wid < _TILES - 1)
    def _full_load():
        pltpu.sync_copy(tid_hbm.at[pl.ds(base, _HPT)], tidv)
        pltpu.sync_copy(beta_hbm.at[pl.ds(base, _HPT)], betav)

    @pl.when(wid == _TILES - 1)
    def _tail_load():
        pltpu.sync_copy(tid_hbm.at[pl.ds(base, _LAST)],
                        tidv.at[pl.ds(0, _LAST)])
        pltpu.sync_copy(beta_hbm.at[pl.ds(base, _LAST)],
                        betav.at[pl.ds(0, _LAST)])

        def fill(j, c):
            s = _LAST + j * 16
            tidv[pl.ds(s, 16)] = jnp.zeros((16,), jnp.int32)
            betav[pl.ds(s, 16)] = jnp.zeros((16,), jnp.float32)
            return c
        lax.fori_loop(0, (_HPT - _LAST) // 16, fill, 0)

    def init_body(j, c):
        s = j * 16
        tbl_b[pl.ds(s, 16)] = jnp.full((16,), -1.0, jnp.float32)
        tbl_c[pl.ds(s, 16)] = jnp.zeros((16,), jnp.float32)
        tbl_a[pl.ds(s, 16)] = jnp.full((16,), _NBIG, jnp.int32)
        return c
    lax.fori_loop(0, _K, init_body, 0)

    def pass_a(i, c):
        off = i * 16
        tv = tidv[pl.ds(off, 16)]
        bv = betav[pl.ds(off, 16)]
        idx = lane_off + tv
        cb = plsc.load_gather(tbl_b, [idx])
        plsc.store_scatter(tbl_b, [idx], jnp.maximum(cb, bv))
        cc = plsc.load_gather(tbl_c, [idx])
        plsc.store_scatter(tbl_c, [idx], cc + 1.0)
        return c
    lax.fori_loop(0, _G, pass_a, 0)

    def lmerge_bc(j, c):
        s = j * 16
        accb = tbl_b[pl.ds(s, 16)]
        accc = tbl_c[pl.ds(s, 16)]
        for t in range(1, 16):
            accb = jnp.maximum(accb, tbl_b[pl.ds(t * _K + s, 16)])
            accc = accc + tbl_c[pl.ds(t * _K + s, 16)]
        g_b[pl.ds(s, 16)] = accb
        g_c[pl.ds(s, 16)] = accc
        return c
    lax.fori_loop(0, 16, lmerge_bc, 0)

    pltpu.sync_copy(g_b, sh_b.at[pl.ds(wid * _K, _K)])
    pltpu.sync_copy(g_c, sh_c.at[pl.ds(wid * _K, _K)])
    plsc.subcore_barrier()

    pltpu.sync_copy(sh_b, stage)

    def gmerge_b(j, c):
        s = j * 16
        acc = stage[pl.ds(s, 16)]
        for t in range(1, _TILES):
            acc = jnp.maximum(acc, stage[pl.ds(t * _K + s, 16)])
        g_b[pl.ds(s, 16)] = acc
        return c
    lax.fori_loop(0, 16, gmerge_b, 0)

    pltpu.sync_copy(sh_c, stage)

    def gmerge_c(j, c):
        s = j * 16
        acc = stage[pl.ds(s, 16)]
        for t in range(1, _TILES):
            acc = acc + stage[pl.ds(t * _K + s, 16)]
        g_c[pl.ds(s, 16)] = acc
        return c
    lax.fori_loop(0, 16, gmerge_c, 0)

    def pass_b1(i, c):
        off = i * 16
        tv = tidv[pl.ds(off, 16)]
        bv = betav[pl.ds(off, 16)]
        gb = plsc.load_gather(g_b, [tv])
        gidx = base + off + lane
        idx = lane_off + tv
        cur = plsc.load_gather(tbl_a, [idx])
        new = jnp.where(bv == gb, jnp.minimum(cur, gidx), cur)
        plsc.store_scatter(tbl_a, [idx], new)
        return c
    lax.fori_loop(0, _G, pass_b1, 0)

    def lmerge_a(j, c):
        s = j * 16
        acc = tbl_a[pl.ds(s, 16)]
        for t in range(1, 16):
            acc = jnp.minimum(acc, tbl_a[pl.ds(t * _K + s, 16)])
        g_a[pl.ds(s, 16)] = acc
        return c
    lax.fori_loop(0, 16, lmerge_a, 0)

    pltpu.sync_copy(g_a, sh_a.at[pl.ds(wid * _K, _K)])
    plsc.subcore_barrier()
    pltpu.sync_copy(sh_a, stagei)

    def gmerge_a(j, c):
        s = j * 16
        acc = stagei[pl.ds(s, 16)]
        for t in range(1, _TILES):
            acc = jnp.minimum(acc, stagei[pl.ds(t * _K + s, 16)])
        g_a[pl.ds(s, 16)] = acc
        aclip[pl.ds(s, 16)] = jnp.minimum(acc, _N - 1)
        return c
    lax.fori_loop(0, 16, gmerge_a, 0)

    pltpu.async_copy(xs_hbm.at[aclip], g_ax, sem).wait()
    pltpu.async_copy(ys_hbm.at[aclip], g_ay, sem).wait()

    def pass_b2(i, c):
        off = i * 16
        tv = tidv[pl.ds(off, 16)]
        axnv[pl.ds(off, 16)] = plsc.load_gather(g_ax, [tv])
        aynv[pl.ds(off, 16)] = plsc.load_gather(g_ay, [tv])
        bmnv[pl.ds(off, 16)] = plsc.load_gather(g_b, [tv])
        cnnv[pl.ds(off, 16)] = plsc.load_gather(g_c, [tv])
        return c
    lax.fori_loop(0, _G, pass_b2, 0)

    pltpu.sync_copy(axnv, axn_o.at[pl.ds(base, _HPT)])
    pltpu.sync_copy(aynv, ayn_o.at[pl.ds(base, _HPT)])
    pltpu.sync_copy(bmnv, bmn_o.at[pl.ds(base, _HPT)])
    pltpu.sync_copy(cnnv, cnn_o.at[pl.ds(base, _HPT)])

    @pl.when(wid == 0)
    def _emit_tables():
        pltpu.sync_copy(g_b, bmax_o)
        pltpu.sync_copy(g_c, cnt_o)
        pltpu.sync_copy(g_ax, ax_o)
        pltpu.sync_copy(g_ay, ay_o)


def _sc_call(tid_raw, beta_raw, xs, ys):
    f32 = jnp.float32
    kern = pl.kernel(
        _sc_body,
        out_type=[
            jax.ShapeDtypeStruct((_K,), f32),
            jax.ShapeDtypeStruct((_K,), f32),
            jax.ShapeDtypeStruct((_K,), f32),
            jax.ShapeDtypeStruct((_K,), f32),
            jax.ShapeDtypeStruct((_NP,), f32),
            jax.ShapeDtypeStruct((_NP,), f32),
            jax.ShapeDtypeStruct((_NP,), f32),
            jax.ShapeDtypeStruct((_NP,), f32),
        ],
        mesh=plsc.VectorSubcoreMesh(
            core_axis_name="c", subcore_axis_name="s",
            num_cores=1, num_subcores=_TILES),
        compiler_params=pltpu.CompilerParams(needs_layout_passes=False),
        scratch_types=[
            pltpu.VMEM((_HPT,), jnp.int32),
            pltpu.VMEM((_HPT,), f32),
            pltpu.VMEM((16 * _K,), f32),
            pltpu.VMEM((16 * _K,), f32),
            pltpu.VMEM((16 * _K,), jnp.int32),
            pltpu.VMEM((_K,), f32),
            pltpu.VMEM((_K,), f32),
            pltpu.VMEM((_K,), jnp.int32),
            pltpu.VMEM((_K,), jnp.int32),
            pltpu.VMEM((_K,), f32),
            pltpu.VMEM((_K,), f32),
            pltpu.VMEM((_TILES * _K,), f32),
            pltpu.VMEM((_TILES * _K,), jnp.int32),
            pltpu.VMEM((_HPT,), f32),
            pltpu.VMEM((_HPT,), f32),
            pltpu.VMEM((_HPT,), f32),
            pltpu.VMEM((_HPT,), f32),
            pltpu.VMEM_SHARED((_TILES * _K,), f32),
            pltpu.VMEM_SHARED((_TILES * _K,), f32),
            pltpu.VMEM_SHARED((_TILES * _K,), jnp.int32),
            pltpu.SemaphoreType.DMA,
        ],
    )
    return kern(tid_raw, beta_raw, xs, ys)


def _tc_body(cc_ref, beta_ref, tid_ref, pe_ref, te_ref, xs_ref, ys_ref,
             axn_ref, ayn_ref, bmn_ref, cnn_ref,
             ax_ref, ay_ref, bmax_ref, cntk_ref,
             out_ref, rep_ref, scal_ref):
    i = pl.program_id(0)

    @pl.when(i == 0)
    def _init():
        rep_ref[...] = jnp.zeros_like(rep_ref)
        scal_ref[...] = jnp.zeros_like(scal_ref)

    gidx = i * _BN + lax.broadcasted_iota(jnp.int32, (1, _BN), 1)
    real = gidx < _N
    beta = jnp.clip(beta_ref[...], 0.0, 1.0 - 1e-4)
    ath = 0.5 * jnp.log((1.0 + beta) / (1.0 - beta))
    q = jnp.where(real, ath * ath + _QMIN, 0.0)

    bf16 = jnp.bfloat16
    real_col = (i * _BN + lax.broadcasted_iota(jnp.int32, (_BN, 1), 0)) < _N
    xs_col = cc_ref[:, 0:1].astype(bf16)
    ys_col = cc_ref[:, 1:2].astype(bf16)
    dx = xs_col - ax_ref[...].astype(bf16)
    dy = ys_col - ay_ref[...].astype(bf16)
    dist = jnp.sqrt(dx * dx + dy * dy + bf16(1e-6))
    hinge = jnp.where(real_col, jnp.maximum(bf16(0.0), bf16(1.0) - dist),
                      bf16(0.0))
    rep_ref[...] += jax.lax.dot_general(
        q.astype(bf16), hinge, (((1,), (0,)), ((), ())),
        preferred_element_type=jnp.float32)

    tid = tid_ref[...]
    cnn = cnn_ref[...]
    ownb = real & (tid > 0) & (cnn > 0.0)
    bmn = jnp.clip(bmn_ref[...], 0.0, 1.0 - 1e-4)
    athn = 0.5 * jnp.log((1.0 + bmn) / (1.0 - bmn))
    qan = athn * athn + _QMIN
    dxo = xs_ref[...] - axn_ref[...]
    dyo = ys_ref[...] - ayn_ref[...]
    dsq_own = dxo * dxo + dyo * dyo
    hinge_own = jnp.maximum(0.0, 1.0 - jnp.sqrt(dsq_own + 1e-6))
    qq = q * qan
    att_s = jnp.sum(jnp.where(ownb, qq * dsq_own / (cnn + 1e-3), 0.0))
    rep_own_s = jnp.sum(jnp.where(ownb, qq * hinge_own, 0.0))

    is_noise = (tid == 0) & real
    not_noise = (tid != 0) & real
    pe = pe_ref[...]
    te = te_ref[...]
    eph = (pe - te) ** 2 / (te + 1.0)
    s0 = jnp.sum(jnp.where(is_noise, beta, 0.0))
    s1 = jnp.sum(is_noise.astype(jnp.float32))
    s2 = jnp.sum(jnp.where(not_noise, eph, 0.0))

    lanev = lax.broadcasted_iota(jnp.int32, (1, 128), 1)
    upd = jnp.where(lanev == 0, att_s,
          jnp.where(lanev == 1, rep_own_s,
          jnp.where(lanev == 2, s0,
          jnp.where(lanev == 3, s1,
          jnp.where(lanev == 4, s2, 0.0)))))
    scal_ref[...] += upd

    @pl.when(i == _GRID - 1)
    def _fin():
        kio = lax.broadcasted_iota(jnp.int32, (1, _K), 1)
        cntk = cntk_ref[...]
        validk = ((kio > 0) & (cntk > 0.0)).astype(jnp.float32)
        bmk = jnp.clip(bmax_ref[...], 0.0, 1.0 - 1e-4)
        athk = 0.5 * jnp.log((1.0 + bmk) / (1.0 - bmk))
        w_rep = (athk * athk + _QMIN) * validk
        n_obj = jnp.sum(validk) + 1e-3
        att_t = scal_ref[0, 0]
        rep_own_t = scal_ref[0, 1]
        t0 = scal_ref[0, 2]
        t1 = scal_ref[0, 3]
        t2 = scal_ref[0, 4]
        v_att = att_t / n_obj
        rep_total = jnp.sum(rep_ref[...] * w_rep) - rep_own_t
        v_rep = rep_total / (float(_N) * n_obj)
        beta_pen = jnp.sum((1.0 - bmax_ref[...]) * validk) / n_obj
        noise_pen = _SB * t0 / (t1 + 1e-3)
        eloss = t2 / ((float(_N) - t1) + 1e-3)
        loss = v_att + v_rep + beta_pen + noise_pen + eloss
        lv = lax.broadcasted_iota(jnp.int32, (1, 128), 1)
        out_ref[...] = jnp.where(lv == 0, loss, 0.0)


def _tc_call(cc, beta_r, tid_r, pe_r, te_r, xs_r, ys_r,
             axn_r, ayn_r, bmn_r, cnn_r, ax2, ay2, bmax2, cnt2):
    row_n = pl.BlockSpec((1, _BN), lambda i: (0, i))
    row_p = pl.BlockSpec((1, _BN), lambda i: (0, i))
    kc = pl.BlockSpec((1, _K), lambda i: (0, 0))
    return pl.pallas_call(
        _tc_body,
        grid=(_GRID,),
        in_specs=[pl.BlockSpec((_BN, 2), lambda i: (i, 0)),
                  row_n, row_n, row_n, row_n, row_n, row_n,
                  row_p, row_p, row_p, row_p,
                  kc, kc, kc, kc],
        out_specs=pl.BlockSpec((1, 128), lambda i: (0, 0)),
        out_shape=jax.ShapeDtypeStruct((1, 128), jnp.float32),
        scratch_shapes=[
            pltpu.VMEM((1, _K), jnp.float32),
            pltpu.VMEM((1, 128), jnp.float32),
        ],
    )(cc, beta_r, tid_r, pe_r, te_r, xs_r, ys_r,
      axn_r, ayn_r, bmn_r, cnn_r, ax2, ay2, bmax2, cnt2)


def kernel(pred_beta, pred_ccoords, pred_energy, t_idx, t_energy):
    tid_raw = t_idx[:, 0]
    beta_raw = pred_beta[:, 0]
    xs = pred_ccoords[:, 0]
    ys = pred_ccoords[:, 1]

    bmax, counts, ax, ay, axn, ayn, bmn, cnn = _sc_call(
        tid_raw, beta_raw, xs, ys)

    out = _tc_call(
        pred_ccoords,
        beta_raw[None, :], tid_raw[None, :],
        pred_energy[:, 0][None, :], t_energy[:, 0][None, :],
        xs[None, :], ys[None, :],
        axn[None, :], ayn[None, :], bmn[None, :], cnn[None, :],
        ax[None, :], ay[None, :], bmax[None, :], counts[None, :])
    return out[0, 0]
